# baseline (speedup 1.0000x reference)
import functools

import jax
import jax.numpy as jnp
from jax import lax
from jax.experimental import pallas as pl
from jax.experimental.pallas import tpu as pltpu
from jax.experimental.pallas import tpu_sc as plsc

SEQ = 2048
DIM = 2048
HID = 5632
T = 512
TE = 256
CAP = SEQ + T
NT = CAP // T
NTB = CAP // TE
H = 512
NH = HID // H
DIMW = DIM // 2
NC, NS, L = 2, 16, 16
NW = NC * NS
RPW = CAP // NW
CH = 16
NCH = RPW // CH
PAD_DST = SEQ

_mesh = plsc.VectorSubcoreMesh(
    core_axis_name="c", subcore_axis_name="s", num_cores=NC, num_subcores=NS)


@functools.partial(
    pl.kernel,
    out_type=(jax.ShapeDtypeStruct((CAP,), jnp.int32),
              jax.ShapeDtypeStruct((CAP,), jnp.int32),
              jax.ShapeDtypeStruct((16,), jnp.int32)),
    mesh=_mesh,
    scratch_types=[pltpu.VMEM((SEQ,), jnp.int32),
                   pltpu.VMEM((CAP,), jnp.int32),
                   pltpu.VMEM((CAP,), jnp.int32),
                   pltpu.VMEM((16,), jnp.int32)],
    compiler_params=pltpu.CompilerParams(needs_layout_passes=False),
)
def _route(mask_hbm, gidx_hbm, sidx_hbm, texp_hbm, mask_v, gidx_v, sidx_v,
           texp_v):
    wid = lax.axis_index("s") * NC + lax.axis_index("c")

    @pl.when(wid == 0)
    def _():
        pltpu.sync_copy(mask_hbm, mask_v)
        iota = lax.iota(jnp.int32, L)

        def count_body(i, n0):
            return n0 + jnp.sum(mask_v[pl.ds(i * L, L)])

        n0 = lax.fori_loop(0, SEQ // L, count_body, jnp.int32(0))
        r0 = ((n0 + T - 1) // T) * T

        def init_body(i, carry):
            gidx_v[pl.ds(i * L, L)] = jnp.zeros((L,), jnp.int32)
            sidx_v[pl.ds(i * L, L)] = PAD_DST + (iota & 7)
            return carry

        lax.fori_loop(0, CAP // L, init_body, 0)

        def scat_body(i, carry):
            c0, c1 = carry
            m = mask_v[pl.ds(i * L, L)]
            inc0 = plsc.cumsum(m)
            inc1 = plsc.cumsum(1 - m)
            dst = jnp.where(m == 1, c0 + inc0 - 1, r0 + c1 + inc1 - 1)
            tok = iota + i * L
            plsc.store_scatter(gidx_v, [dst], tok)
            plsc.store_scatter(sidx_v, [dst], tok)
            return (c0 + jnp.sum(m), c1 + jnp.sum(1 - m))

        lax.fori_loop(0, SEQ // L, scat_body,
                      (jnp.int32(0), jnp.int32(0)))

        texp_v[...] = jnp.where(iota * TE >= r0, 1, 0).astype(jnp.int32)
        pltpu.sync_copy(gidx_v, gidx_hbm)
        pltpu.sync_copy(sidx_v, sidx_hbm)
        pltpu.sync_copy(texp_v, texp_hbm)


@functools.partial(
    pl.kernel,
    out_type=jax.ShapeDtypeStruct((CAP, DIM), jnp.float32),
    mesh=_mesh,
    scratch_types=[pltpu.VMEM((RPW,), jnp.int32),
                   pltpu.VMEM((2, CH, DIM), jnp.float32),
                   pltpu.SemaphoreType.DMA,
                   pltpu.SemaphoreType.DMA],
)
def _gather(x_hbm, gidx_hbm, xs_hbm, idx_v, rows_v, sem0, sem1):
    wid = lax.axis_index("s") * NC + lax.axis_index("c")
    base = wid * RPW
    pltpu.sync_copy(gidx_hbm.at[pl.ds(base, RPW)], idx_v)
    sems = [sem0, sem1]
    hs = [None] * NCH
    hs[0] = pltpu.async_copy(x_hbm.at[idx_v.at[pl.ds(0, CH)]],
                             rows_v.at[0], sems[0])
    hs[1] = pltpu.async_copy(x_hbm.at[idx_v.at[pl.ds(CH, CH)]],
                             rows_v.at[1], sems[1])
    for c in range(NCH):
        hs[c].wait()
        pltpu.sync_copy(rows_v.at[c % 2], xs_hbm.at[pl.ds(base + c * CH, CH)])
        if c + 2 < NCH:
            hs[c + 2] = pltpu.async_copy(
                x_hbm.at[idx_v.at[pl.ds((c + 2) * CH, CH)]],
                rows_v.at[c % 2], sems[c % 2])


@functools.partial(
    pl.kernel,
    out_type=jax.ShapeDtypeStruct((SEQ + 8, DIM), jnp.float32),
    mesh=_mesh,
    scratch_types=[pltpu.VMEM((NCH, CH), jnp.int32),
                   pltpu.VMEM((2, CH, DIM), jnp.float32),
                   pltpu.SemaphoreType.DMA,
                   pltpu.SemaphoreType.DMA],
)
def _scatter(ys_hbm, sidx_hbm, out_hbm, idx_v, rows_v, sem0, sem1):
    wid = lax.axis_index("s") * NC + lax.axis_index("c")
    base = wid * RPW
    pltpu.sync_copy(sidx_hbm.at[wid], idx_v)
    sems = [sem0, sem1]
    prev = None
    for c in range(NCH):
        pltpu.sync_copy(ys_hbm.at[pl.ds(base + c * CH, CH)],
                        rows_v.at[c % 2])
        if prev is not None:
            prev.wait()
        prev = pltpu.async_copy(rows_v.at[c % 2], out_hbm.at[idx_v.at[c]],
                                sems[c % 2])
    prev.wait()


def _hold1(h, e_t):
    return jnp.maximum(h - 1 + e_t, 0)


def _ffn_a_body(e_ref, x_ref, w10, w30, w11, w31, hh_ref, xbf, w1b, w3b):
    h = pl.program_id(0)
    t = pl.program_id(1)
    et = e_ref[t * (T // TE)]
    eprev = e_ref[jnp.maximum(t - 1, 0) * (T // TE)]
    sl = pl.ds(t * T, T)
    dn = (((1,), (1,)), ((), ()))

    @pl.when(h == 0)
    def _():
        xbf[sl, :] = x_ref[...].astype(jnp.bfloat16)

    @pl.when((t == 0) | (et != eprev))
    def _():
        @pl.when(et == 0)
        def _():
            w1b[...] = w10[...].astype(jnp.bfloat16)
            w3b[...] = w30[...].astype(jnp.bfloat16)

        @pl.when(et != 0)
        def _():
            w1b[...] = w11[...].astype(jnp.bfloat16)
            w3b[...] = w31[...].astype(jnp.bfloat16)

    a = lax.dot_general(xbf[sl, :], w1b[...], dn,
                        preferred_element_type=jnp.float32)
    b = lax.dot_general(xbf[sl, :], w3b[...], dn,
                        preferred_element_type=jnp.float32)
    hh_ref[...] = a * lax.logistic(a) * b


_ffn_a = pl.pallas_call(
    _ffn_a_body,
    grid_spec=pltpu.PrefetchScalarGridSpec(
        num_scalar_prefetch=1,
        grid=(NH, NT),
        in_specs=[
            pl.BlockSpec((T, DIM),
                         lambda h, t, e: (jnp.where(h == 0, t, NT - 1), 0)),
            pl.BlockSpec((H, DIM), lambda h, t, e: (h, 0)),
            pl.BlockSpec((H, DIM), lambda h, t, e: (h, 0)),
            pl.BlockSpec((H, DIM), lambda h, t, e: (_hold1(h, e[t * (T // TE)]), 0)),
            pl.BlockSpec((H, DIM), lambda h, t, e: (_hold1(h, e[t * (T // TE)]), 0)),
        ],
        out_specs=pl.BlockSpec((T, H), lambda h, t, e: (t, h)),
        scratch_shapes=[pltpu.VMEM((CAP, DIM), jnp.bfloat16),
                        pltpu.VMEM((H, DIM), jnp.bfloat16),
                        pltpu.VMEM((H, DIM), jnp.bfloat16)],
    ),
    out_shape=jax.ShapeDtypeStruct((CAP, HID), jnp.float32),
    compiler_params=pltpu.CompilerParams(
        dimension_semantics=("arbitrary", "arbitrary"),
        vmem_limit_bytes=60 * 1024 * 1024,
    ),
)


DIMH = DIM // 2


def _ffn_b_body(e_ref, hh_ref, w2s, out_ref):
    out_ref[...] = lax.dot_general(hh_ref[...].astype(jnp.bfloat16), w2s[0],
                                   (((1,), (1,)), ((), ())),
                                   preferred_element_type=jnp.float32)


_ffn_b = pl.pallas_call(
    _ffn_b_body,
    grid_spec=pltpu.PrefetchScalarGridSpec(
        num_scalar_prefetch=1,
        grid=(2, NT),
        in_specs=[
            pl.BlockSpec((T, HID), lambda d, t, e: (t, 0)),
            pl.BlockSpec((1, DIMH, HID), lambda d, t, e: (e[t * (T // TE)], d, 0)),
        ],
        out_specs=pl.BlockSpec((T, DIMH), lambda d, t, e: (t, d)),
    ),
    out_shape=jax.ShapeDtypeStruct((CAP, DIM), jnp.float32),
    compiler_params=pltpu.CompilerParams(
        dimension_semantics=("arbitrary", "arbitrary"),
        vmem_limit_bytes=60 * 1024 * 1024,
    ),
)


def kernel(x, modality_masks, w1_0, w2_0, w3_0, w1_1, w2_1, w3_1):
    x2d = x.reshape(SEQ, DIM)
    mask0 = modality_masks[0].reshape(SEQ).astype(jnp.int32)
    gidx, sidx, texp = _route(mask0)
    xs = _gather(x2d, gidx)
    hh = _ffn_a(texp, xs, w1_0, w3_0, w1_1, w3_1)
    w2s_bf = jnp.stack([w2_0, w2_1]).astype(jnp.bfloat16)
    ys = _ffn_b(texp, hh, w2s_bf)
    merged_pad = _scatter(ys, sidx.reshape(NW, NCH, CH))
    return merged_pad[:SEQ].reshape(1, SEQ, DIM)


# speedup vs baseline: 1.1035x; 1.1035x over previous
import functools

import jax
import jax.numpy as jnp
from jax import lax
from jax.experimental import pallas as pl
from jax.experimental.pallas import tpu as pltpu
from jax.experimental.pallas import tpu_sc as plsc

SEQ = 2048
DIM = 2048
HID = 5632
T = 512
TE = 256
CAP = SEQ + T
NT = CAP // T
NTB = CAP // TE
H = 512
NH = HID // H
DIMW = DIM // 2
NC, NS, L = 2, 16, 16
NW = NC * NS
RPW = CAP // NW
CH = 16
NCH = RPW // CH
PAD_DST = SEQ

_mesh = plsc.VectorSubcoreMesh(
    core_axis_name="c", subcore_axis_name="s", num_cores=NC, num_subcores=NS)


@functools.partial(
    pl.kernel,
    out_type=(jax.ShapeDtypeStruct((CAP,), jnp.int32),
              jax.ShapeDtypeStruct((CAP,), jnp.int32),
              jax.ShapeDtypeStruct((16,), jnp.int32)),
    mesh=_mesh,
    scratch_types=[pltpu.VMEM((SEQ,), jnp.int32),
                   pltpu.VMEM((CAP,), jnp.int32),
                   pltpu.VMEM((CAP,), jnp.int32),
                   pltpu.VMEM((16,), jnp.int32)],
    compiler_params=pltpu.CompilerParams(needs_layout_passes=False),
)
def _route(mask_hbm, gidx_hbm, sidx_hbm, texp_hbm, mask_v, gidx_v, sidx_v,
           texp_v):
    wid = lax.axis_index("s") * NC + lax.axis_index("c")

    @pl.when(wid == 0)
    def _():
        pltpu.sync_copy(mask_hbm, mask_v)
        iota = lax.iota(jnp.int32, L)

        def count_body(i, n0):
            return n0 + jnp.sum(mask_v[pl.ds(i * L, L)])

        n0 = lax.fori_loop(0, SEQ // L, count_body, jnp.int32(0))
        r0 = ((n0 + T - 1) // T) * T

        def init_body(i, carry):
            gidx_v[pl.ds(i * L, L)] = jnp.zeros((L,), jnp.int32)
            sidx_v[pl.ds(i * L, L)] = PAD_DST + (iota & 7)
            return carry

        lax.fori_loop(0, CAP // L, init_body, 0)

        def scat_body(i, carry):
            c0, c1 = carry
            m = mask_v[pl.ds(i * L, L)]
            inc0 = plsc.cumsum(m)
            inc1 = plsc.cumsum(1 - m)
            dst = jnp.where(m == 1, c0 + inc0 - 1, r0 + c1 + inc1 - 1)
            tok = iota + i * L
            plsc.store_scatter(gidx_v, [dst], tok)
            plsc.store_scatter(sidx_v, [dst], tok)
            return (c0 + jnp.sum(m), c1 + jnp.sum(1 - m))

        lax.fori_loop(0, SEQ // L, scat_body,
                      (jnp.int32(0), jnp.int32(0)))

        texp_v[...] = jnp.where(iota * TE >= r0, 1, 0).astype(jnp.int32)
        pltpu.sync_copy(gidx_v, gidx_hbm)
        pltpu.sync_copy(sidx_v, sidx_hbm)
        pltpu.sync_copy(texp_v, texp_hbm)


@functools.partial(
    pl.kernel,
    out_type=jax.ShapeDtypeStruct((CAP, DIM), jnp.float32),
    mesh=_mesh,
    scratch_types=[pltpu.VMEM((RPW,), jnp.int32),
                   pltpu.VMEM((2, CH, DIM), jnp.float32),
                   pltpu.SemaphoreType.DMA,
                   pltpu.SemaphoreType.DMA],
)
def _gather(x_hbm, gidx_hbm, xs_hbm, idx_v, rows_v, sem0, sem1):
    wid = lax.axis_index("s") * NC + lax.axis_index("c")
    base = wid * RPW
    pltpu.sync_copy(gidx_hbm.at[pl.ds(base, RPW)], idx_v)
    sems = [sem0, sem1]
    hs = [None] * NCH
    hs[0] = pltpu.async_copy(x_hbm.at[idx_v.at[pl.ds(0, CH)]],
                             rows_v.at[0], sems[0])
    hs[1] = pltpu.async_copy(x_hbm.at[idx_v.at[pl.ds(CH, CH)]],
                             rows_v.at[1], sems[1])
    for c in range(NCH):
        hs[c].wait()
        pltpu.sync_copy(rows_v.at[c % 2], xs_hbm.at[pl.ds(base + c * CH, CH)])
        if c + 2 < NCH:
            hs[c + 2] = pltpu.async_copy(
                x_hbm.at[idx_v.at[pl.ds((c + 2) * CH, CH)]],
                rows_v.at[c % 2], sems[c % 2])


@functools.partial(
    pl.kernel,
    out_type=jax.ShapeDtypeStruct((SEQ + 8, DIM), jnp.float32),
    mesh=_mesh,
    scratch_types=[pltpu.VMEM((NCH, CH), jnp.int32),
                   pltpu.VMEM((2, CH, DIM), jnp.float32),
                   pltpu.SemaphoreType.DMA,
                   pltpu.SemaphoreType.DMA],
)
def _scatter(ys_hbm, sidx_hbm, out_hbm, idx_v, rows_v, sem0, sem1):
    wid = lax.axis_index("s") * NC + lax.axis_index("c")
    base = wid * RPW
    pltpu.sync_copy(sidx_hbm.at[wid], idx_v)
    sems = [sem0, sem1]
    prev = None
    for c in range(NCH):
        pltpu.sync_copy(ys_hbm.at[pl.ds(base + c * CH, CH)],
                        rows_v.at[c % 2])
        if prev is not None:
            prev.wait()
        prev = pltpu.async_copy(rows_v.at[c % 2], out_hbm.at[idx_v.at[c]],
                                sems[c % 2])
    prev.wait()


def _hold1(h, e_t):
    return jnp.maximum(h - 1 + e_t, 0)


def _ffn_a_body(e_ref, x_ref, w10, w30, w11, w31, hh_ref):
    t = pl.program_id(1)
    et = e_ref[t * (T // TE)]
    sl = pl.ds(t * T, T)
    dn = (((1,), (1,)), ((), ()))

    def compute(w1r, w3r):
        x = x_ref[sl, :]
        a = lax.dot_general(x, w1r[...], dn,
                            preferred_element_type=jnp.float32)
        b = lax.dot_general(x, w3r[...], dn,
                            preferred_element_type=jnp.float32)
        hh_ref[...] = (a * lax.logistic(a) * b).astype(jnp.bfloat16)

    @pl.when(et == 0)
    def _():
        compute(w10, w30)

    @pl.when(et != 0)
    def _():
        compute(w11, w31)


_ffn_a = pl.pallas_call(
    _ffn_a_body,
    grid_spec=pltpu.PrefetchScalarGridSpec(
        num_scalar_prefetch=1,
        grid=(NH, NT),
        in_specs=[
            pl.BlockSpec((CAP, DIM), lambda h, t, e: (0, 0)),
            pl.BlockSpec((H, DIM), lambda h, t, e: (h, 0)),
            pl.BlockSpec((H, DIM), lambda h, t, e: (h, 0)),
            pl.BlockSpec((H, DIM), lambda h, t, e: (_hold1(h, e[t * (T // TE)]), 0)),
            pl.BlockSpec((H, DIM), lambda h, t, e: (_hold1(h, e[t * (T // TE)]), 0)),
        ],
        out_specs=pl.BlockSpec((T, H), lambda h, t, e: (t, h)),
    ),
    out_shape=jax.ShapeDtypeStruct((CAP, HID), jnp.bfloat16),
    compiler_params=pltpu.CompilerParams(
        dimension_semantics=("arbitrary", "arbitrary"),
        vmem_limit_bytes=60 * 1024 * 1024,
    ),
)


def _ffn_b_body(e_ref, hh_ref, w20, w21, out_ref):
    t = pl.program_id(0)
    et = e_ref[t]
    dn = (((1,), (1,)), ((), ()))

    def compute(w2r):
        out_ref[...] = lax.dot_general(hh_ref[...], w2r[...], dn,
                                       preferred_element_type=jnp.float32)

    @pl.when(et == 0)
    def _():
        compute(w20)

    @pl.when(et != 0)
    def _():
        compute(w21)


_ffn_b = pl.pallas_call(
    _ffn_b_body,
    grid_spec=pltpu.PrefetchScalarGridSpec(
        num_scalar_prefetch=1,
        grid=(NTB,),
        in_specs=[
            pl.BlockSpec((TE, HID), lambda t, e: (t, 0)),
            pl.BlockSpec((DIM, HID), lambda t, e: (0, 0)),
            pl.BlockSpec((DIM, HID), lambda t, e: (0, 0)),
        ],
        out_specs=pl.BlockSpec((TE, DIM), lambda t, e: (t, 0)),
    ),
    out_shape=jax.ShapeDtypeStruct((CAP, DIM), jnp.float32),
    compiler_params=pltpu.CompilerParams(
        dimension_semantics=("arbitrary",),
        vmem_limit_bytes=60 * 1024 * 1024,
    ),
)


def kernel(x, modality_masks, w1_0, w2_0, w3_0, w1_1, w2_1, w3_1):
    x2d = x.reshape(SEQ, DIM)
    mask0 = modality_masks[0].reshape(SEQ).astype(jnp.int32)
    gidx, sidx, texp = _route(mask0)
    xs = _gather(x2d, gidx)
    hh = _ffn_a(texp, xs, w1_0, w3_0, w1_1, w3_1)
    ys = _ffn_b(texp, hh, w2_0.astype(jnp.bfloat16),
                w2_1.astype(jnp.bfloat16))
    merged_pad = _scatter(ys, sidx.reshape(NW, NCH, CH))
    return merged_pad[:SEQ].reshape(1, SEQ, DIM)


# speedup vs baseline: 1.1098x; 1.0057x over previous
import functools

import jax
import jax.numpy as jnp
from jax import lax
from jax.experimental import pallas as pl
from jax.experimental.pallas import tpu as pltpu
from jax.experimental.pallas import tpu_sc as plsc

SEQ = 2048
DIM = 2048
HID = 5632
T = 256
TE = 256
CAP = SEQ + T
NT = CAP // T
NTB = CAP // TE
H = 512
NH = HID // H
DIMW = DIM // 2
NC, NS, L = 2, 16, 16
NW = NC * NS
RPW = CAP // NW
CH = 24
NCH = RPW // CH
PAD_DST = SEQ

_mesh = plsc.VectorSubcoreMesh(
    core_axis_name="c", subcore_axis_name="s", num_cores=NC, num_subcores=NS)


@functools.partial(
    pl.kernel,
    out_type=(jax.ShapeDtypeStruct((CAP,), jnp.int32),
              jax.ShapeDtypeStruct((CAP,), jnp.int32),
              jax.ShapeDtypeStruct((16,), jnp.int32)),
    mesh=_mesh,
    scratch_types=[pltpu.VMEM((SEQ,), jnp.int32),
                   pltpu.VMEM((CAP,), jnp.int32),
                   pltpu.VMEM((CAP,), jnp.int32),
                   pltpu.VMEM((16,), jnp.int32)],
    compiler_params=pltpu.CompilerParams(needs_layout_passes=False),
)
def _route(mask_hbm, gidx_hbm, sidx_hbm, texp_hbm, mask_v, gidx_v, sidx_v,
           texp_v):
    wid = lax.axis_index("s") * NC + lax.axis_index("c")

    @pl.when(wid == 0)
    def _():
        pltpu.sync_copy(mask_hbm, mask_v)
        iota = lax.iota(jnp.int32, L)

        def count_body(i, n0):
            return n0 + jnp.sum(mask_v[pl.ds(i * L, L)])

        n0 = lax.fori_loop(0, SEQ // L, count_body, jnp.int32(0))
        r0 = ((n0 + T - 1) // T) * T

        def init_body(i, carry):
            gidx_v[pl.ds(i * L, L)] = jnp.zeros((L,), jnp.int32)
            sidx_v[pl.ds(i * L, L)] = PAD_DST + (iota & 7)
            return carry

        lax.fori_loop(0, CAP // L, init_body, 0)

        def scat_body(i, carry):
            c0, c1 = carry
            m = mask_v[pl.ds(i * L, L)]
            inc0 = plsc.cumsum(m)
            inc1 = plsc.cumsum(1 - m)
            dst = jnp.where(m == 1, c0 + inc0 - 1, r0 + c1 + inc1 - 1)
            tok = iota + i * L
            plsc.store_scatter(gidx_v, [dst], tok)
            plsc.store_scatter(sidx_v, [dst], tok)
            return (c0 + jnp.sum(m), c1 + jnp.sum(1 - m))

        lax.fori_loop(0, SEQ // L, scat_body,
                      (jnp.int32(0), jnp.int32(0)))

        texp_v[...] = jnp.where(iota * TE >= r0, 1, 0).astype(jnp.int32)
        pltpu.sync_copy(gidx_v, gidx_hbm)
        pltpu.sync_copy(sidx_v, sidx_hbm)
        pltpu.sync_copy(texp_v, texp_hbm)


@functools.partial(
    pl.kernel,
    out_type=jax.ShapeDtypeStruct((CAP, DIM), jnp.float32),
    mesh=_mesh,
    scratch_types=[pltpu.VMEM((RPW,), jnp.int32),
                   pltpu.VMEM((2, CH, DIM), jnp.float32),
                   pltpu.SemaphoreType.DMA,
                   pltpu.SemaphoreType.DMA],
)
def _gather(x_hbm, gidx_hbm, xs_hbm, idx_v, rows_v, sem0, sem1):
    wid = lax.axis_index("s") * NC + lax.axis_index("c")
    base = wid * RPW
    pltpu.sync_copy(gidx_hbm.at[pl.ds(base, RPW)], idx_v)
    sems = [sem0, sem1]
    hs = [None] * NCH
    hs[0] = pltpu.async_copy(x_hbm.at[idx_v.at[pl.ds(0, CH)]],
                             rows_v.at[0], sems[0])
    hs[1] = pltpu.async_copy(x_hbm.at[idx_v.at[pl.ds(CH, CH)]],
                             rows_v.at[1], sems[1])
    for c in range(NCH):
        hs[c].wait()
        pltpu.sync_copy(rows_v.at[c % 2], xs_hbm.at[pl.ds(base + c * CH, CH)])
        if c + 2 < NCH:
            hs[c + 2] = pltpu.async_copy(
                x_hbm.at[idx_v.at[pl.ds((c + 2) * CH, CH)]],
                rows_v.at[c % 2], sems[c % 2])


@functools.partial(
    pl.kernel,
    out_type=jax.ShapeDtypeStruct((SEQ + 8, DIM), jnp.float32),
    mesh=_mesh,
    scratch_types=[pltpu.VMEM((NCH, CH), jnp.int32),
                   pltpu.VMEM((2, CH, DIM), jnp.float32),
                   pltpu.SemaphoreType.DMA,
                   pltpu.SemaphoreType.DMA],
)
def _scatter(ys_hbm, sidx_hbm, out_hbm, idx_v, rows_v, sem0, sem1):
    wid = lax.axis_index("s") * NC + lax.axis_index("c")
    base = wid * RPW
    pltpu.sync_copy(sidx_hbm.at[wid], idx_v)
    sems = [sem0, sem1]
    prev = None
    for c in range(NCH):
        pltpu.sync_copy(ys_hbm.at[pl.ds(base + c * CH, CH)],
                        rows_v.at[c % 2])
        if prev is not None:
            prev.wait()
        prev = pltpu.async_copy(rows_v.at[c % 2], out_hbm.at[idx_v.at[c]],
                                sems[c % 2])
    prev.wait()


def _hold1(h, e_t):
    return jnp.maximum(h - 1 + e_t, 0)


def _ffn_a_body(e_ref, x_ref, w10, w30, w11, w31, hh_ref):
    t = pl.program_id(1)
    et = e_ref[t * (T // TE)]
    sl = pl.ds(t * T, T)
    dn = (((1,), (1,)), ((), ()))

    def compute(w1r, w3r):
        x = x_ref[sl, :]
        a = lax.dot_general(x, w1r[...], dn,
                            preferred_element_type=jnp.float32)
        b = lax.dot_general(x, w3r[...], dn,
                            preferred_element_type=jnp.float32)
        hh_ref[...] = (a * lax.logistic(a) * b).astype(jnp.bfloat16)

    @pl.when(et == 0)
    def _():
        compute(w10, w30)

    @pl.when(et != 0)
    def _():
        compute(w11, w31)


_ffn_a = pl.pallas_call(
    _ffn_a_body,
    grid_spec=pltpu.PrefetchScalarGridSpec(
        num_scalar_prefetch=1,
        grid=(NH, NT),
        in_specs=[
            pl.BlockSpec((CAP, DIM), lambda h, t, e: (0, 0)),
            pl.BlockSpec((H, DIM), lambda h, t, e: (h, 0)),
            pl.BlockSpec((H, DIM), lambda h, t, e: (h, 0)),
            pl.BlockSpec((H, DIM), lambda h, t, e: (_hold1(h, e[t * (T // TE)]), 0)),
            pl.BlockSpec((H, DIM), lambda h, t, e: (_hold1(h, e[t * (T // TE)]), 0)),
        ],
        out_specs=pl.BlockSpec((T, H), lambda h, t, e: (t, h)),
    ),
    out_shape=jax.ShapeDtypeStruct((CAP, HID), jnp.bfloat16),
    compiler_params=pltpu.CompilerParams(
        dimension_semantics=("arbitrary", "arbitrary"),
        vmem_limit_bytes=60 * 1024 * 1024,
    ),
)


def _ffn_b_body(e_ref, hh_ref, w20, w21, out_ref):
    t = pl.program_id(0)
    et = e_ref[t]
    dn = (((1,), (1,)), ((), ()))

    def compute(w2r):
        out_ref[...] = lax.dot_general(hh_ref[...], w2r[...], dn,
                                       preferred_element_type=jnp.float32)

    @pl.when(et == 0)
    def _():
        compute(w20)

    @pl.when(et != 0)
    def _():
        compute(w21)


_ffn_b = pl.pallas_call(
    _ffn_b_body,
    grid_spec=pltpu.PrefetchScalarGridSpec(
        num_scalar_prefetch=1,
        grid=(NTB,),
        in_specs=[
            pl.BlockSpec((TE, HID), lambda t, e: (t, 0)),
            pl.BlockSpec((DIM, HID), lambda t, e: (0, 0)),
            pl.BlockSpec((DIM, HID), lambda t, e: (0, 0)),
        ],
        out_specs=pl.BlockSpec((TE, DIM), lambda t, e: (t, 0)),
    ),
    out_shape=jax.ShapeDtypeStruct((CAP, DIM), jnp.float32),
    compiler_params=pltpu.CompilerParams(
        dimension_semantics=("arbitrary",),
        vmem_limit_bytes=60 * 1024 * 1024,
    ),
)


def kernel(x, modality_masks, w1_0, w2_0, w3_0, w1_1, w2_1, w3_1):
    x2d = x.reshape(SEQ, DIM)
    mask0 = modality_masks[0].reshape(SEQ).astype(jnp.int32)
    gidx, sidx, texp = _route(mask0)
    xs = _gather(x2d, gidx)
    hh = _ffn_a(texp, xs, w1_0, w3_0, w1_1, w3_1)
    ys = _ffn_b(texp, hh, w2_0.astype(jnp.bfloat16),
                w2_1.astype(jnp.bfloat16))
    merged_pad = _scatter(ys, sidx.reshape(NW, NCH, CH))
    return merged_pad[:SEQ].reshape(1, SEQ, DIM)
